# baseline (speedup 1.0000x reference)
import functools

import jax
import jax.numpy as jnp
from jax import lax
from jax.experimental import pallas as pl
from jax.experimental.pallas import tpu as pltpu
from jax.experimental.pallas import tpu_sc as plsc

NUM_CORES = 2
NUM_SUBCORES = 16
LANES = 16
NW = NUM_CORES * NUM_SUBCORES

CHUNK = 128


def _offset_kernel(table2, phys, par):
    batch = phys.shape[0] // 3
    b_per_w = batch // NW
    chunks_per_w = b_per_w // CHUNK
    mesh = plsc.VectorSubcoreMesh(core_axis_name="c", subcore_axis_name="s")

    @functools.partial(
        pl.kernel,
        out_type=jax.ShapeDtypeStruct((batch // 2, 128), jnp.float32),
        mesh=mesh,
        scratch_types=[
            pltpu.VMEM((b_per_w,), jnp.int32),
            pltpu.VMEM((b_per_w,), jnp.int32),
            pltpu.VMEM((b_per_w,), jnp.int32),
            pltpu.VMEM((b_per_w,), jnp.int32),
            pltpu.VMEM((b_per_w,), jnp.int32),
            pltpu.VMEM((b_per_w,), jnp.int32),
            pltpu.VMEM((CHUNK, 128), jnp.float32),
            pltpu.VMEM((CHUNK, 128), jnp.float32),
            pltpu.VMEM((CHUNK, 128), jnp.float32),
            pltpu.VMEM((CHUNK // 2, 128), jnp.float32),
            pltpu.SemaphoreType.DMA,
        ],
        compiler_params=pltpu.CompilerParams(use_tc_tiling_on_sc=True),
    )
    def k(table_hbm, phys_hbm, par_hbm, out_hbm,
          i1_v, i2_v, i4_v, p1_v, p2_v, p4_v, a_v, b_v, c_v, o_v, sem):
        wid = lax.axis_index("s") * NUM_CORES + lax.axis_index("c")
        base = wid * b_per_w
        ibase = wid * (3 * b_per_w)
        pltpu.sync_copy(phys_hbm.at[pl.ds(ibase, b_per_w)], i1_v)
        pltpu.sync_copy(phys_hbm.at[pl.ds(ibase + b_per_w, b_per_w)], i2_v)
        pltpu.sync_copy(phys_hbm.at[pl.ds(ibase + 2 * b_per_w, b_per_w)], i4_v)
        pltpu.sync_copy(par_hbm.at[pl.ds(ibase, b_per_w)], p1_v)
        pltpu.sync_copy(par_hbm.at[pl.ds(ibase + b_per_w, b_per_w)], p2_v)
        pltpu.sync_copy(par_hbm.at[pl.ds(ibase + 2 * b_per_w, b_per_w)], p4_v)

        @pl.loop(0, chunks_per_w)
        def _(g):
            off = g * CHUNK
            ca = pltpu.async_copy(
                table_hbm.at[i1_v.at[pl.ds(off, CHUNK)]], a_v, sem)
            cb = pltpu.async_copy(
                table_hbm.at[i2_v.at[pl.ds(off, CHUNK)]], b_v, sem)
            cc = pltpu.async_copy(
                table_hbm.at[i4_v.at[pl.ds(off, CHUNK)]], c_v, sem)
            ca.wait()
            cb.wait()
            cc.wait()

            @pl.loop(0, CHUNK // LANES)
            def _(rg):
                rbase = rg * LANES
                p1 = p1_v[pl.ds(off + rbase, LANES)]
                p2 = p2_v[pl.ds(off + rbase, LANES)]
                p4 = p4_v[pl.ds(off + rbase, LANES)]
                for j in range(LANES):
                    r = rbase + j
                    o1 = p1[j]
                    o2 = p2[j]
                    o4 = p4[j]
                    r2 = rg * (LANES // 2) + j // 2
                    rl = (j % 2) * 64
                    for c in range(0, 64, LANES):
                        o_v[r2, pl.ds(rl + c, LANES)] = (
                            a_v[r, pl.ds(o1 + c, LANES)]
                            - b_v[r, pl.ds(o2 + c, LANES)]
                            + c_v[r, pl.ds(o4 + c, LANES)]
                        )

            obase = pl.multiple_of((base + off) // 2, CHUNK // 2)
            pltpu.sync_copy(o_v, out_hbm.at[pl.ds(obase, CHUNK // 2)])

    return k(table2, phys, par)


def kernel(inputs, table):
    e1 = inputs[:, 0]
    e2 = inputs[:, 1]
    e3 = inputs[:, 2]
    e4 = inputs[:, 3]
    batch = inputs.shape[0]
    idx3 = jnp.stack([e1, e2, e4], axis=0)
    idx3 = idx3.reshape(3, NW, -1).transpose(1, 0, 2).reshape(-1)
    phys = idx3 >> 1
    par = (idx3 & 1) * 64
    table2 = table.reshape(table.shape[0] // 2, 128)
    packed = _offset_kernel(table2, phys, par)
    offset_trick = packed.reshape(batch, 64)
    filters = jnp.concatenate(
        [e1.reshape(-1, 1), e2.reshape(-1, 1), e4.reshape(-1, 1)], axis=1)
    return (e1, e2, e3, e4, offset_trick, filters)
